# baseline (speedup 1.0000x reference)
import functools

import jax
import jax.numpy as jnp
from jax import lax
from jax.experimental import pallas as pl
from jax.experimental.pallas import tpu as pltpu
from jax.experimental.pallas import tpu_sc as plsc

BATCH = 4
SEQ_LEN = 8192
EMBED_DIM = 768
POOL_SIZE = 64
LENGTH = 5
TOP_K = 8
TASK_PROMPT_SIZE = 8

SEQ_BLK = 1024
N_SEQ_BLK = SEQ_LEN // SEQ_BLK
NBLK = BATCH * N_SEQ_BLK
NBUF = 12
LOOKAHEAD = 6
HEAD_ROWS = (TASK_PROMPT_SIZE + TOP_K) * LENGTH
ASSIST_ROWS = TASK_PROMPT_SIZE * LENGTH
SEL_ROWS = TOP_K * LENGTH
GATHER_ROWS = 48
OUT_ROWS = HEAD_ROWS + SEQ_LEN


def _stream(x_hbm, pk_ref, out_hbm, xnorm_ref, sim_ref,
            buf, acc_ref, rsem, wsem):
    b = pl.program_id(0)
    s = pl.program_id(1)
    i = b * N_SEQ_BLK + s

    def read_cp(j):
        bb = j // N_SEQ_BLK
        ss = j - bb * N_SEQ_BLK
        return pltpu.make_async_copy(
            x_hbm.at[pl.ds(bb, 1), pl.ds(ss * SEQ_BLK, SEQ_BLK), :],
            buf.at[pl.ds(j % NBUF, 1)],
            rsem.at[j % NBUF],
        )

    def write_cp(j):
        bb = j // N_SEQ_BLK
        ss = j - bb * N_SEQ_BLK
        return pltpu.make_async_copy(
            buf.at[pl.ds(j % NBUF, 1)],
            out_hbm.at[pl.ds(bb, 1),
                       pl.ds(HEAD_ROWS + ss * SEQ_BLK, SEQ_BLK), :],
            wsem.at[j % NBUF],
        )

    @pl.when(i == 0)
    def _():
        for j in range(LOOKAHEAD):
            read_cp(j).start()

    read_cp(i).wait()
    write_cp(i).start()
    psum = jnp.sum(buf[i % NBUF], axis=0, keepdims=True)

    @pl.when(s == 0)
    def _():
        acc_ref[0:1, :] = psum

    @pl.when(s > 0)
    def _():
        acc_ref[0:1, :] = acc_ref[0:1, :] + psum

    @pl.when(i >= NBUF - LOOKAHEAD)
    def _():
        write_cp(i - (NBUF - LOOKAHEAD)).wait()

    @pl.when(i + LOOKAHEAD < NBLK)
    def _():
        read_cp(i + LOOKAHEAD).start()

    @pl.when(s == N_SEQ_BLK - 1)
    def _():
        mean = acc_ref[0:1, :] * (1.0 / SEQ_LEN)
        ss = jnp.sum(mean * mean, axis=1, keepdims=True)
        xn = mean * jax.lax.rsqrt(jnp.maximum(ss, 1e-12))
        xnorm_ref[pl.ds(b, 1), :] = xn

        pk = pk_ref[:, :]
        pss = jnp.sum(pk * pk, axis=1, keepdims=True)
        pn = pk * jax.lax.rsqrt(jnp.maximum(pss, 1e-12))
        sim_ref[pl.ds(b, 1), :] = jax.lax.dot_general(
            xn, pn, (((1,), (1,)), ((), ())),
            preferred_element_type=jnp.float32,
        )

    @pl.when((b == BATCH - 1) & (s == N_SEQ_BLK - 1))
    def _():
        for k in range(NBUF - LOOKAHEAD):
            write_cp(NBLK - 1 - k).wait()


def _sc_routing(sim_hbm, pr_hbm, ar_hbm, idx_out, bmp_out,
                sim_v, skeys, svals, mk, mv, fvbuf, idx16, gidx,
                rows_v, ar_v, sem):
    wid = lax.axis_index("s") * 2 + lax.axis_index("c")
    iota = lax.iota(jnp.int32, 16)
    sel8 = iota + (iota // 8) * 8

    for b in range(BATCH):
        @pl.when(wid == b)
        def _(b=b):
            pltpu.sync_copy(sim_hbm.at[pl.ds(b, 1)], sim_v)
            pltpu.sync_copy(ar_hbm, ar_v)

            for c in range(POOL_SIZE // 16):
                k = sim_v[0, pl.ds(c * 16, 16)]
                sk, sv = plsc.sort_key_val(k, iota + c * 16, descending=True)
                skeys[pl.ds(c * 16, 16)] = sk
                svals[pl.ds(c * 16, 16)] = sv
            for h in range(2):
                ck = plsc.load_gather(skeys, [sel8 + 32 * h])
                cv = plsc.load_gather(svals, [sel8 + 32 * h])
                sk, sv = plsc.sort_key_val(ck, cv, descending=True)
                mk[pl.ds(h * 16, 16)] = sk
                mv[pl.ds(h * 16, 16)] = sv
            ck = plsc.load_gather(mk, [sel8])
            cv = plsc.load_gather(mv, [sel8])
            _, fv = plsc.sort_key_val(ck, cv, descending=True)
            idx16[0, :] = fv
            fvbuf[:] = fv

            for c in range(3):
                j = iota + c * 16
                p = j // LENGTH
                sel = plsc.load_gather(fvbuf, [p])
                gidx[pl.ds(c * 16, 16)] = sel * LENGTH + (j - p * LENGTH)

            pltpu.async_copy(pr_hbm.at[gidx], rows_v, sem).wait()
            pltpu.sync_copy(ar_v, bmp_out.at[b, pl.ds(0, ASSIST_ROWS)])
            pltpu.sync_copy(rows_v.at[pl.ds(0, SEL_ROWS)],
                            bmp_out.at[b, pl.ds(ASSIST_ROWS, SEL_ROWS)])
            pltpu.sync_copy(idx16, idx_out.at[pl.ds(b, 1)])


def _epilogue(bmp_ref, sim_ref, idx_ref, pe_in, pe_out, rsum_ref, sem):
    iota64 = jax.lax.broadcasted_iota(
        jnp.int32, (BATCH, POOL_SIZE), 1).astype(jnp.float32)
    idxf = idx_ref[:, :].astype(jnp.float32)
    cacc = jnp.zeros((BATCH, POOL_SIZE), jnp.float32)
    for k in range(TOP_K):
        cacc = cacc + (idxf[:, k:k + 1] == iota64).astype(jnp.float32)
    counts = jnp.sum(cacc, axis=0, keepdims=True)
    colsum = jnp.sum(sim_ref[:, :], axis=0, keepdims=True)
    rsum_ref[0, 0] = jnp.sum(counts * colsum) * (1.0 / BATCH)

    cp = pltpu.make_async_copy(
        bmp_ref, pe_out.at[:, pl.ds(0, HEAD_ROWS), :], sem)
    cp.start()
    cp.wait()


def kernel(x_embed, prompt, prompt_key, assist_prompt, test=1, threshold=-2):
    prompt_r = prompt.reshape(POOL_SIZE * LENGTH, EMBED_DIM)
    assist_r = assist_prompt.reshape(ASSIST_ROWS, EMBED_DIM)

    prompted, xnorm, sim = pl.pallas_call(
        _stream,
        grid=(BATCH, N_SEQ_BLK),
        in_specs=[
            pl.BlockSpec(memory_space=pl.ANY),
            pl.BlockSpec((POOL_SIZE, EMBED_DIM), lambda b, s: (0, 0)),
        ],
        out_specs=[
            pl.BlockSpec(memory_space=pl.ANY),
            pl.BlockSpec((BATCH, EMBED_DIM), lambda b, s: (0, 0)),
            pl.BlockSpec((BATCH, POOL_SIZE), lambda b, s: (0, 0)),
        ],
        out_shape=[
            jax.ShapeDtypeStruct((BATCH, OUT_ROWS, EMBED_DIM), jnp.float32),
            jax.ShapeDtypeStruct((BATCH, EMBED_DIM), jnp.float32),
            jax.ShapeDtypeStruct((BATCH, POOL_SIZE), jnp.float32),
        ],
        scratch_shapes=[
            pltpu.VMEM((NBUF, SEQ_BLK, EMBED_DIM), jnp.float32),
            pltpu.VMEM((8, EMBED_DIM), jnp.float32),
            pltpu.SemaphoreType.DMA((NBUF,)),
            pltpu.SemaphoreType.DMA((NBUF,)),
        ],
    )(x_embed, prompt_key)

    sc_routing = functools.partial(
        pl.kernel,
        out_type=[
            jax.ShapeDtypeStruct((BATCH, 16), jnp.int32),
            jax.ShapeDtypeStruct((BATCH, HEAD_ROWS, EMBED_DIM), jnp.float32),
        ],
        mesh=plsc.VectorSubcoreMesh(core_axis_name="c", subcore_axis_name="s"),
        compiler_params=pltpu.CompilerParams(needs_layout_passes=False),
        scratch_types=[
            pltpu.VMEM((1, POOL_SIZE), jnp.float32),
            pltpu.VMEM((POOL_SIZE,), jnp.float32),
            pltpu.VMEM((POOL_SIZE,), jnp.int32),
            pltpu.VMEM((32,), jnp.float32),
            pltpu.VMEM((32,), jnp.int32),
            pltpu.VMEM((16,), jnp.int32),
            pltpu.VMEM((1, 16), jnp.int32),
            pltpu.VMEM((GATHER_ROWS,), jnp.int32),
            pltpu.VMEM((GATHER_ROWS, EMBED_DIM), jnp.float32),
            pltpu.VMEM((ASSIST_ROWS, EMBED_DIM), jnp.float32),
            pltpu.SemaphoreType.DMA,
        ],
    )(_sc_routing)
    idx16, bmp = sc_routing(sim, prompt_r, assist_r)
    idx = idx16[:, :TOP_K]

    prompted, rsum = pl.pallas_call(
        _epilogue,
        in_specs=[
            pl.BlockSpec((BATCH, HEAD_ROWS, EMBED_DIM), lambda: (0, 0, 0)),
            pl.BlockSpec((BATCH, POOL_SIZE), lambda: (0, 0)),
            pl.BlockSpec((BATCH, TOP_K), lambda: (0, 0)),
            pl.BlockSpec(memory_space=pl.ANY),
        ],
        out_specs=[
            pl.BlockSpec(memory_space=pl.ANY),
            pl.BlockSpec(memory_space=pltpu.SMEM),
        ],
        out_shape=[
            jax.ShapeDtypeStruct((BATCH, OUT_ROWS, EMBED_DIM), jnp.float32),
            jax.ShapeDtypeStruct((1, 1), jnp.float32),
        ],
        input_output_aliases={3: 0},
        scratch_shapes=[pltpu.SemaphoreType.DMA],
    )(bmp, sim, idx, prompted)

    return prompted, rsum.reshape(()), bmp, xnorm, idx


# speedup vs baseline: 1.0254x; 1.0254x over previous
import functools

import jax
import jax.numpy as jnp
from jax import lax
from jax.experimental import pallas as pl
from jax.experimental.pallas import tpu as pltpu
from jax.experimental.pallas import tpu_sc as plsc

BATCH = 4
SEQ_LEN = 8192
EMBED_DIM = 768
POOL_SIZE = 64
LENGTH = 5
TOP_K = 8
TASK_PROMPT_SIZE = 8

SEQ_BLK = 1024
N_SEQ_BLK = SEQ_LEN // SEQ_BLK
NBLK = BATCH * N_SEQ_BLK
NBUF = 12
LOOKAHEAD = 6
HEAD_ROWS = (TASK_PROMPT_SIZE + TOP_K) * LENGTH
ASSIST_ROWS = TASK_PROMPT_SIZE * LENGTH
SEL_ROWS = TOP_K * LENGTH
GATHER_ROWS = 48
OUT_ROWS = HEAD_ROWS + SEQ_LEN


def _stream(x_hbm, pk_ref, out_hbm, xnorm_ref, sim_ref,
            buf, acc_ref, rsem, wsem):
    b = pl.program_id(0)
    s = pl.program_id(1)
    i = b * N_SEQ_BLK + s

    def read_cp(j):
        bb = j // N_SEQ_BLK
        ss = j - bb * N_SEQ_BLK
        return pltpu.make_async_copy(
            x_hbm.at[pl.ds(bb, 1), pl.ds(ss * SEQ_BLK, SEQ_BLK), :],
            buf.at[pl.ds(j % NBUF, 1)],
            rsem.at[j % NBUF],
        )

    def write_cp(j):
        bb = j // N_SEQ_BLK
        ss = j - bb * N_SEQ_BLK
        return pltpu.make_async_copy(
            buf.at[pl.ds(j % NBUF, 1)],
            out_hbm.at[pl.ds(bb, 1),
                       pl.ds(HEAD_ROWS + ss * SEQ_BLK, SEQ_BLK), :],
            wsem.at[j % NBUF],
        )

    @pl.when(i == 0)
    def _():
        for j in range(LOOKAHEAD):
            read_cp(j).start()

    read_cp(i).wait()
    write_cp(i).start()
    psum = jnp.sum(buf[i % NBUF], axis=0, keepdims=True)

    @pl.when(s == 0)
    def _():
        acc_ref[0:1, :] = psum

    @pl.when(s > 0)
    def _():
        acc_ref[0:1, :] = acc_ref[0:1, :] + psum

    @pl.when(i >= NBUF - LOOKAHEAD)
    def _():
        write_cp(i - (NBUF - LOOKAHEAD)).wait()

    @pl.when(i + LOOKAHEAD < NBLK)
    def _():
        read_cp(i + LOOKAHEAD).start()

    @pl.when(s == N_SEQ_BLK - 1)
    def _():
        mean = acc_ref[0:1, :] * (1.0 / SEQ_LEN)
        ss = jnp.sum(mean * mean, axis=1, keepdims=True)
        xn = mean * jax.lax.rsqrt(jnp.maximum(ss, 1e-12))
        xnorm_ref[pl.ds(b, 1), :] = xn

        pk = pk_ref[:, :]
        pss = jnp.sum(pk * pk, axis=1, keepdims=True)
        pn = pk * jax.lax.rsqrt(jnp.maximum(pss, 1e-12))
        sim_ref[pl.ds(b, 1), :] = jax.lax.dot_general(
            xn, pn, (((1,), (1,)), ((), ())),
            preferred_element_type=jnp.float32,
        )

    @pl.when((b == BATCH - 1) & (s == N_SEQ_BLK - 1))
    def _():
        for k in range(NBUF - LOOKAHEAD):
            write_cp(NBLK - 1 - k).wait()


def _sc_routing(sim_hbm, pr_hbm, ar_hbm, idx_out, bmp_out,
                sim_v, skeys, svals, mk, mv, fvbuf, idx16, gidx,
                rows_v, ar_v, sem):
    wid = lax.axis_index("s") * 2 + lax.axis_index("c")
    iota = lax.iota(jnp.int32, 16)
    sel8 = iota + (iota // 8) * 8
    AR_OFF = (0, 8, 16, 24)
    AR_CNT = (8, 8, 8, 16)

    for b in range(BATCH):
        for w in range(4):
            @pl.when(wid == b * 4 + w)
            def _(b=b, w=w):
                pltpu.sync_copy(sim_hbm.at[pl.ds(b, 1)], sim_v)

                for c in range(POOL_SIZE // 16):
                    k = sim_v[0, pl.ds(c * 16, 16)]
                    sk, sv = plsc.sort_key_val(k, iota + c * 16,
                                               descending=True)
                    skeys[pl.ds(c * 16, 16)] = sk
                    svals[pl.ds(c * 16, 16)] = sv
                for h in range(2):
                    ck = plsc.load_gather(skeys, [sel8 + 32 * h])
                    cv = plsc.load_gather(svals, [sel8 + 32 * h])
                    sk, sv = plsc.sort_key_val(ck, cv, descending=True)
                    mk[pl.ds(h * 16, 16)] = sk
                    mv[pl.ds(h * 16, 16)] = sv
                ck = plsc.load_gather(mk, [sel8])
                cv = plsc.load_gather(mv, [sel8])
                _, fv = plsc.sort_key_val(ck, cv, descending=True)

                pltpu.sync_copy(ar_hbm.at[pl.ds(AR_OFF[w], AR_CNT[w])],
                                ar_v.at[pl.ds(0, AR_CNT[w])])
                pltpu.sync_copy(
                    ar_v.at[pl.ds(0, AR_CNT[w])],
                    bmp_out.at[b, pl.ds(AR_OFF[w], AR_CNT[w])])

                if w < 3:
                    fvbuf[:] = fv
                    j = iota + w * 16
                    p = j // LENGTH
                    sel = plsc.load_gather(fvbuf, [p])
                    gidx[:] = sel * LENGTH + (j - p * LENGTH)
                    pltpu.async_copy(pr_hbm.at[gidx], rows_v, sem).wait()
                    nrows = 16 if w < 2 else SEL_ROWS - 32
                    pltpu.sync_copy(
                        rows_v.at[pl.ds(0, nrows)],
                        bmp_out.at[b, pl.ds(ASSIST_ROWS + 16 * w, nrows)])
                else:
                    idx16[0, :] = fv
                    pltpu.sync_copy(idx16, idx_out.at[pl.ds(b, 1)])


def _epilogue(bmp_ref, sim_ref, idx_ref, pe_in, pe_out, rsum_ref, sem):
    iota64 = jax.lax.broadcasted_iota(
        jnp.int32, (BATCH, POOL_SIZE), 1).astype(jnp.float32)
    idxf = idx_ref[:, :].astype(jnp.float32)
    cacc = jnp.zeros((BATCH, POOL_SIZE), jnp.float32)
    for k in range(TOP_K):
        cacc = cacc + (idxf[:, k:k + 1] == iota64).astype(jnp.float32)
    counts = jnp.sum(cacc, axis=0, keepdims=True)
    colsum = jnp.sum(sim_ref[:, :], axis=0, keepdims=True)
    rsum_ref[0, 0] = jnp.sum(counts * colsum) * (1.0 / BATCH)

    cp = pltpu.make_async_copy(
        bmp_ref, pe_out.at[:, pl.ds(0, HEAD_ROWS), :], sem)
    cp.start()
    cp.wait()


def kernel(x_embed, prompt, prompt_key, assist_prompt, test=1, threshold=-2):
    prompt_r = prompt.reshape(POOL_SIZE * LENGTH, EMBED_DIM)
    assist_r = assist_prompt.reshape(ASSIST_ROWS, EMBED_DIM)

    prompted, xnorm, sim = pl.pallas_call(
        _stream,
        grid=(BATCH, N_SEQ_BLK),
        in_specs=[
            pl.BlockSpec(memory_space=pl.ANY),
            pl.BlockSpec((POOL_SIZE, EMBED_DIM), lambda b, s: (0, 0)),
        ],
        out_specs=[
            pl.BlockSpec(memory_space=pl.ANY),
            pl.BlockSpec((BATCH, EMBED_DIM), lambda b, s: (0, 0)),
            pl.BlockSpec((BATCH, POOL_SIZE), lambda b, s: (0, 0)),
        ],
        out_shape=[
            jax.ShapeDtypeStruct((BATCH, OUT_ROWS, EMBED_DIM), jnp.float32),
            jax.ShapeDtypeStruct((BATCH, EMBED_DIM), jnp.float32),
            jax.ShapeDtypeStruct((BATCH, POOL_SIZE), jnp.float32),
        ],
        scratch_shapes=[
            pltpu.VMEM((NBUF, SEQ_BLK, EMBED_DIM), jnp.float32),
            pltpu.VMEM((8, EMBED_DIM), jnp.float32),
            pltpu.SemaphoreType.DMA((NBUF,)),
            pltpu.SemaphoreType.DMA((NBUF,)),
        ],
    )(x_embed, prompt_key)

    sc_routing = functools.partial(
        pl.kernel,
        out_type=[
            jax.ShapeDtypeStruct((BATCH, 16), jnp.int32),
            jax.ShapeDtypeStruct((BATCH, HEAD_ROWS, EMBED_DIM), jnp.float32),
        ],
        mesh=plsc.VectorSubcoreMesh(core_axis_name="c", subcore_axis_name="s"),
        compiler_params=pltpu.CompilerParams(needs_layout_passes=False),
        scratch_types=[
            pltpu.VMEM((1, POOL_SIZE), jnp.float32),
            pltpu.VMEM((POOL_SIZE,), jnp.float32),
            pltpu.VMEM((POOL_SIZE,), jnp.int32),
            pltpu.VMEM((32,), jnp.float32),
            pltpu.VMEM((32,), jnp.int32),
            pltpu.VMEM((16,), jnp.int32),
            pltpu.VMEM((1, 16), jnp.int32),
            pltpu.VMEM((16,), jnp.int32),
            pltpu.VMEM((16, EMBED_DIM), jnp.float32),
            pltpu.VMEM((16, EMBED_DIM), jnp.float32),
            pltpu.SemaphoreType.DMA,
        ],
    )(_sc_routing)
    idx16, bmp = sc_routing(sim, prompt_r, assist_r)
    idx = idx16[:, :TOP_K]

    prompted, rsum = pl.pallas_call(
        _epilogue,
        in_specs=[
            pl.BlockSpec((BATCH, HEAD_ROWS, EMBED_DIM), lambda: (0, 0, 0)),
            pl.BlockSpec((BATCH, POOL_SIZE), lambda: (0, 0)),
            pl.BlockSpec((BATCH, TOP_K), lambda: (0, 0)),
            pl.BlockSpec(memory_space=pl.ANY),
        ],
        out_specs=[
            pl.BlockSpec(memory_space=pl.ANY),
            pl.BlockSpec(memory_space=pltpu.SMEM),
        ],
        out_shape=[
            jax.ShapeDtypeStruct((BATCH, OUT_ROWS, EMBED_DIM), jnp.float32),
            jax.ShapeDtypeStruct((1, 1), jnp.float32),
        ],
        input_output_aliases={3: 0},
        scratch_shapes=[pltpu.SemaphoreType.DMA],
    )(bmp, sim, idx, prompted)

    return prompted, rsum.reshape(()), bmp, xnorm, idx
